# baseline (speedup 1.0000x reference)
import functools

import jax
import jax.numpy as jnp
from jax import lax
from jax.experimental import pallas as pl
from jax.experimental.pallas import tpu as pltpu
from jax.experimental.pallas import tpu_sc as plsc

NC, NS, LANES = 2, 16, 16
NW = NC * NS

B, L, D = 4, 8192, 768
N = B * L
TOK_PER_W = N // NW
CHUNK = 32
NCHUNK = TOK_PER_W // CHUNK
NJ = D // LANES
NACC = 4
LN_EPS = 1e-12


def _allsum16(v):
    iota = lax.iota(jnp.int32, LANES)
    for shift in (8, 4, 2, 1):
        idx = (iota + shift) & (LANES - 1)
        v = v + v.at[idx].get(mode="promise_in_bounds")
    return v


def _rsqrt16(v):
    i = lax.bitcast_convert_type(v, jnp.int32)
    i = jnp.int32(0x5F3759DF) - lax.shift_right_logical(i, 1)
    y = lax.bitcast_convert_type(i, jnp.float32)
    for _ in range(2):
        y = y * (1.5 - 0.5 * v * y * y)
    return y


def _sc_embed(ids, pos, tt, word_table, pos_table, type_table, ln_w, ln_b):
    mesh = plsc.VectorSubcoreMesh(core_axis_name="c", subcore_axis_name="s")

    @functools.partial(
        pl.kernel,
        mesh=mesh,
        out_type=jax.ShapeDtypeStruct((N, D), jnp.float32),
        scratch_types=[
            pltpu.VMEM((CHUNK,), jnp.int32),
            pltpu.VMEM((CHUNK,), jnp.int32),
            pltpu.VMEM((CHUNK,), jnp.int32),
            pltpu.VMEM((CHUNK,), jnp.int32),
            pltpu.VMEM((CHUNK,), jnp.float32),
            pltpu.VMEM((CHUNK,), jnp.float32),
            pltpu.VMEM((CHUNK,), jnp.int32),
            pltpu.VMEM((CHUNK,), jnp.int32),
            pltpu.VMEM((CHUNK, D), jnp.float32),
            pltpu.VMEM((CHUNK, D), jnp.float32),
            pltpu.VMEM((CHUNK, D), jnp.float32),
            pltpu.VMEM((CHUNK, D), jnp.float32),
            pltpu.VMEM((2, D), jnp.float32),
            pltpu.VMEM((D,), jnp.float32),
            pltpu.VMEM((D,), jnp.float32),
            pltpu.VMEM((D,), jnp.float32),
            pltpu.SemaphoreType.DMA,
            pltpu.SemaphoreType.DMA,
            pltpu.SemaphoreType.DMA,
            pltpu.SemaphoreType.DMA,
        ],
    )
    def k(ids_hbm, pos_hbm, tt_hbm, word_hbm, post_hbm, type_hbm, lnw_hbm,
          lnb_hbm, out_hbm, idw0, idw1, idp0, idp1, ttf0, ttf1, tti0, tti1,
          rw0, rw1, rp0, rp1, ty_v, td_v, lnw_v, lnb_v,
          sw0, sw1, sp0, sp1):
        wid = lax.axis_index("s") * NC + lax.axis_index("c")
        wbase = wid * TOK_PER_W

        idw = (idw0, idw1)
        idp = (idp0, idp1)
        ttf = (ttf0, ttf1)
        tti = (tti0, tti1)
        rw = (rw0, rw1)
        rp = (rp0, rp1)
        sw = (sw0, sw1)
        sp = (sp0, sp1)

        pltpu.sync_copy(type_hbm, ty_v)
        pltpu.sync_copy(lnw_hbm, lnw_v)
        pltpu.sync_copy(lnb_hbm, lnb_v)
        for j in range(NJ):
            sl = pl.ds(j * LANES, LANES)
            td_v[sl] = ty_v[1, sl] - ty_v[0, sl]

        def issue(g, b):
            base = wbase + g * CHUNK
            pltpu.sync_copy(ids_hbm.at[pl.ds(base, CHUNK)], idw[b])
            pltpu.sync_copy(pos_hbm.at[pl.ds(base, CHUNK)], idp[b])
            pltpu.sync_copy(tt_hbm.at[pl.ds(base, CHUNK)], tti[b])
            pltpu.async_copy(word_hbm.at[idw[b]], rw[b], sw[b])
            pltpu.async_copy(post_hbm.at[idp[b]], rp[b], sp[b])

        def wait(b):
            pltpu.make_async_copy(word_hbm.at[idw[b]], rw[b], sw[b]).wait()
            pltpu.make_async_copy(post_hbm.at[idp[b]], rp[b], sp[b]).wait()

        def compute(g, b):
            rw_v, rp_v, ttf_v, tti_v = rw[b], rp[b], ttf[b], tti[b]
            for q in range(CHUNK // LANES):
                qsl = pl.ds(q * LANES, LANES)
                ttf_v[qsl] = tti_v[qsl].astype(jnp.float32)

            def tok_body(t2, tc):
                for u in range(2):
                    t = t2 * 2 + u
                    g16 = t // LANES
                    lane = t - g16 * LANES
                    lane_v = lax.broadcast_in_dim(lane, (LANES,), ())
                    ttb = ttf_v[pl.ds(g16 * LANES, LANES)].at[lane_v].get(
                        mode="promise_in_bounds")
                    acc = [jnp.zeros((LANES,), jnp.float32)
                           for _ in range(NACC)]
                    acc2 = [jnp.zeros((LANES,), jnp.float32)
                            for _ in range(NACC)]
                    for j in range(NJ):
                        sl = pl.ds(j * LANES, LANES)
                        x = rw_v[t, sl] + rp_v[t, sl] + (ty_v[0, sl]
                                                         + ttb * td_v[sl])
                        rw_v[t, sl] = x
                        acc[j % NACC] = acc[j % NACC] + x
                        acc2[j % NACC] = acc2[j % NACC] + x * x
                    a = (acc[0] + acc[1]) + (acc[2] + acc[3])
                    a2 = (acc2[0] + acc2[1]) + (acc2[2] + acc2[3])
                    mv = _allsum16(a) * (1.0 / D)
                    ex2 = _allsum16(a2) * (1.0 / D)
                    var = ex2 - mv * mv
                    inv = _rsqrt16(var + LN_EPS)
                    for j in range(NJ):
                        sl = pl.ds(j * LANES, LANES)
                        xn = (rw_v[t, sl] - mv) * inv
                        rw_v[t, sl] = xn * lnw_v[sl] + lnb_v[sl]
                return tc

            lax.fori_loop(0, CHUNK // 2, tok_body, 0)
            pltpu.sync_copy(rw_v, out_hbm.at[pl.ds(wbase + g * CHUNK, CHUNK)])

        issue(0, 0)

        def pair_body(i, carry):
            g0 = 2 * i
            wait(0)
            issue(g0 + 1, 1)
            compute(g0, 0)
            wait(1)

            @pl.when(i < NCHUNK // 2 - 1)
            def _():
                issue(g0 + 2, 0)

            compute(g0 + 1, 1)
            return carry

        lax.fori_loop(0, NCHUNK // 2, pair_body, 0)

    return k(ids, pos, tt, word_table, pos_table, type_table, ln_w, ln_b)


def kernel(input_ids, token_type_ids, position_ids, word_table, pos_table,
           type_table, ln_w, ln_b):
    ids = jnp.asarray(input_ids, jnp.int32).reshape(N)
    pos = jnp.asarray(position_ids, jnp.int32).reshape(N)
    tt = jnp.asarray(token_type_ids, jnp.int32).reshape(N)
    out = _sc_embed(ids, pos, tt,
                    word_table.astype(jnp.float32),
                    pos_table.astype(jnp.float32),
                    type_table.astype(jnp.float32),
                    ln_w.astype(jnp.float32), ln_b.astype(jnp.float32))
    return out.reshape(B, L, D)


# speedup vs baseline: 2.1372x; 2.1372x over previous
import functools

import jax
import jax.numpy as jnp
from jax import lax
from jax.experimental import pallas as pl
from jax.experimental.pallas import tpu as pltpu
from jax.experimental.pallas import tpu_sc as plsc

NC, NS, LANES = 2, 16, 16
NW = NC * NS

B, L, D = 4, 8192, 768
N = B * L
TOK_PER_W = N // NW
CHUNK = 32
NCHUNK = TOK_PER_W // CHUNK
NJ = D // LANES
NACC = 4
LN_EPS = 1e-12


def _allsum16(v):
    iota = lax.iota(jnp.int32, LANES)
    for shift in (8, 4, 2, 1):
        idx = (iota + shift) & (LANES - 1)
        v = v + v.at[idx].get(mode="promise_in_bounds")
    return v


def _rsqrt16(v):
    i = lax.bitcast_convert_type(v, jnp.int32)
    i = jnp.int32(0x5F3759DF) - lax.shift_right_logical(i, 1)
    y = lax.bitcast_convert_type(i, jnp.float32)
    for _ in range(2):
        y = y * (1.5 - 0.5 * v * y * y)
    return y


def _sc_embed(ids, pos, tt, word_table, pos_table, type_table, ln_w, ln_b):
    mesh = plsc.VectorSubcoreMesh(core_axis_name="c", subcore_axis_name="s")

    @functools.partial(
        pl.kernel,
        mesh=mesh,
        out_type=jax.ShapeDtypeStruct((N, D), jnp.float32),
        scratch_types=[
            pltpu.VMEM((CHUNK,), jnp.int32),
            pltpu.VMEM((CHUNK,), jnp.int32),
            pltpu.VMEM((CHUNK,), jnp.int32),
            pltpu.VMEM((CHUNK,), jnp.int32),
            pltpu.VMEM((CHUNK,), jnp.float32),
            pltpu.VMEM((CHUNK,), jnp.float32),
            pltpu.VMEM((CHUNK,), jnp.int32),
            pltpu.VMEM((CHUNK,), jnp.int32),
            pltpu.VMEM((CHUNK, D), jnp.float32),
            pltpu.VMEM((CHUNK, D), jnp.float32),
            pltpu.VMEM((CHUNK, D), jnp.float32),
            pltpu.VMEM((CHUNK, D), jnp.float32),
            pltpu.VMEM((2, D), jnp.float32),
            pltpu.VMEM((D,), jnp.float32),
            pltpu.VMEM((D,), jnp.float32),
            pltpu.VMEM((D,), jnp.float32),
            pltpu.SemaphoreType.DMA,
            pltpu.SemaphoreType.DMA,
            pltpu.SemaphoreType.DMA,
            pltpu.SemaphoreType.DMA,
        ],
    )
    def k(ids_hbm, pos_hbm, tt_hbm, word_hbm, post_hbm, type_hbm, lnw_hbm,
          lnb_hbm, out_hbm, idw0, idw1, idp0, idp1, ttf0, ttf1, tti0, tti1,
          rw0, rw1, rp0, rp1, ty_v, td_v, lnw_v, lnb_v,
          sw0, sw1, sp0, sp1):
        wid = lax.axis_index("s") * NC + lax.axis_index("c")
        wbase = wid * TOK_PER_W

        idw = (idw0, idw1)
        idp = (idp0, idp1)
        ttf = (ttf0, ttf1)
        tti = (tti0, tti1)
        rw = (rw0, rw1)
        rp = (rp0, rp1)
        sw = (sw0, sw1)
        sp = (sp0, sp1)

        pltpu.sync_copy(type_hbm, ty_v)
        pltpu.sync_copy(lnw_hbm, lnw_v)
        pltpu.sync_copy(lnb_hbm, lnb_v)
        for j in range(NJ):
            sl = pl.ds(j * LANES, LANES)
            td_v[sl] = ty_v[1, sl] - ty_v[0, sl]

        def issue(g, b):
            base = wbase + g * CHUNK
            pltpu.sync_copy(ids_hbm.at[pl.ds(base, CHUNK)], idw[b])
            pltpu.sync_copy(pos_hbm.at[pl.ds(base, CHUNK)], idp[b])
            pltpu.sync_copy(tt_hbm.at[pl.ds(base, CHUNK)], tti[b])
            pltpu.async_copy(word_hbm.at[idw[b]], rw[b], sw[b])
            pltpu.async_copy(post_hbm.at[idp[b]], rp[b], sp[b])

        def wait(b):
            pltpu.make_async_copy(word_hbm.at[idw[b]], rw[b], sw[b]).wait()
            pltpu.make_async_copy(post_hbm.at[idp[b]], rp[b], sp[b]).wait()

        def compute(g, b):
            rw_v, rp_v, ttf_v, tti_v = rw[b], rp[b], ttf[b], tti[b]
            for q in range(CHUNK // LANES):
                qsl = pl.ds(q * LANES, LANES)
                ttf_v[qsl] = tti_v[qsl].astype(jnp.float32)

            def tok_body(t2, tc):
                for u in range(1):
                    t = t2 + u
                    g16 = t // LANES
                    lane = t - g16 * LANES
                    lane_v = lax.broadcast_in_dim(lane, (LANES,), ())
                    ttb = ttf_v[pl.ds(g16 * LANES, LANES)].at[lane_v].get(
                        mode="promise_in_bounds")
                    acc = [jnp.zeros((LANES,), jnp.float32)
                           for _ in range(NACC)]
                    acc2 = [jnp.zeros((LANES,), jnp.float32)
                            for _ in range(NACC)]
                    for j in range(NJ):
                        sl = pl.ds(j * LANES, LANES)
                        x = rw_v[t, sl] + rp_v[t, sl] + (ty_v[0, sl]
                                                         + ttb * td_v[sl])
                        rw_v[t, sl] = x
                        acc[j % NACC] = acc[j % NACC] + x
                        acc2[j % NACC] = acc2[j % NACC] + x * x
                    a = (acc[0] + acc[1]) + (acc[2] + acc[3])
                    a2 = (acc2[0] + acc2[1]) + (acc2[2] + acc2[3])
                    mv = _allsum16(a) * (1.0 / D)
                    ex2 = _allsum16(a2) * (1.0 / D)
                    var = ex2 - mv * mv
                    inv = _rsqrt16(var + LN_EPS)
                    for j in range(NJ):
                        sl = pl.ds(j * LANES, LANES)
                        xn = (rw_v[t, sl] - mv) * inv
                        rw_v[t, sl] = xn * lnw_v[sl] + lnb_v[sl]
                return tc

            lax.fori_loop(0, CHUNK, tok_body, 0)
            pltpu.sync_copy(rw_v, out_hbm.at[pl.ds(wbase + g * CHUNK, CHUNK)])

        issue(0, 0)

        def pair_body(i, carry):
            g0 = 2 * i
            wait(0)
            issue(g0 + 1, 1)
            compute(g0, 0)
            wait(1)

            @pl.when(i < NCHUNK // 2 - 1)
            def _():
                issue(g0 + 2, 0)

            compute(g0 + 1, 1)
            return carry

        lax.fori_loop(0, NCHUNK // 2, pair_body, 0)

    return k(ids, pos, tt, word_table, pos_table, type_table, ln_w, ln_b)


def kernel(input_ids, token_type_ids, position_ids, word_table, pos_table,
           type_table, ln_w, ln_b):
    ids = jnp.asarray(input_ids, jnp.int32).reshape(N)
    pos = jnp.asarray(position_ids, jnp.int32).reshape(N)
    tt = jnp.asarray(token_type_ids, jnp.int32).reshape(N)
    out = _sc_embed(ids, pos, tt,
                    word_table.astype(jnp.float32),
                    pos_table.astype(jnp.float32),
                    type_table.astype(jnp.float32),
                    ln_w.astype(jnp.float32), ln_b.astype(jnp.float32))
    return out.reshape(B, L, D)


# speedup vs baseline: 2.1416x; 1.0021x over previous
import functools

import jax
import jax.numpy as jnp
from jax import lax
from jax.experimental import pallas as pl
from jax.experimental.pallas import tpu as pltpu
from jax.experimental.pallas import tpu_sc as plsc

NC, NS, LANES = 2, 16, 16
NW = NC * NS

B, L, D = 4, 8192, 768
N = B * L
TOK_PER_W = N // NW
CHUNK = 32
NCHUNK = TOK_PER_W // CHUNK
NJ = D // LANES
NACC = 4
LN_EPS = 1e-12


def _allsum16(v):
    iota = lax.iota(jnp.int32, LANES)
    for shift in (8, 4, 2, 1):
        idx = (iota + shift) & (LANES - 1)
        v = v + v.at[idx].get(mode="promise_in_bounds")
    return v


def _rsqrt16(v):
    i = lax.bitcast_convert_type(v, jnp.int32)
    i = jnp.int32(0x5F3759DF) - lax.shift_right_logical(i, 1)
    y = lax.bitcast_convert_type(i, jnp.float32)
    for _ in range(2):
        y = y * (1.5 - 0.5 * v * y * y)
    return y


def _sc_embed(ids, pos, tt, word_table, pos_table, type_table, ln_w, ln_b):
    mesh = plsc.VectorSubcoreMesh(core_axis_name="c", subcore_axis_name="s")

    @functools.partial(
        pl.kernel,
        mesh=mesh,
        out_type=jax.ShapeDtypeStruct((N, D), jnp.float32),
        scratch_types=[
            pltpu.VMEM((CHUNK,), jnp.int32),
            pltpu.VMEM((CHUNK,), jnp.int32),
            pltpu.VMEM((CHUNK,), jnp.int32),
            pltpu.VMEM((CHUNK,), jnp.int32),
            pltpu.VMEM((CHUNK,), jnp.float32),
            pltpu.VMEM((CHUNK,), jnp.float32),
            pltpu.VMEM((CHUNK,), jnp.int32),
            pltpu.VMEM((CHUNK,), jnp.int32),
            pltpu.VMEM((CHUNK, D), jnp.float32),
            pltpu.VMEM((CHUNK, D), jnp.float32),
            pltpu.VMEM((CHUNK, D), jnp.float32),
            pltpu.VMEM((CHUNK, D), jnp.float32),
            pltpu.VMEM((CHUNK, D), jnp.float32),
            pltpu.VMEM((2, D), jnp.float32),
            pltpu.VMEM((D,), jnp.float32),
            pltpu.VMEM((D,), jnp.float32),
            pltpu.VMEM((D,), jnp.float32),
            pltpu.SemaphoreType.DMA,
            pltpu.SemaphoreType.DMA,
            pltpu.SemaphoreType.DMA,
            pltpu.SemaphoreType.DMA,
        ],
    )
    def k(ids_hbm, pos_hbm, tt_hbm, word_hbm, post_hbm, type_hbm, lnw_hbm,
          lnb_hbm, out_hbm, idw0, idw1, idp0, idp1, ttf0, ttf1, tti0, tti1,
          rw0, rw1, rp0, rp1, xv_v, ty_v, td_v, lnw_v, lnb_v,
          sw0, sw1, sp0, sp1):
        wid = lax.axis_index("s") * NC + lax.axis_index("c")
        wbase = wid * TOK_PER_W

        idw = (idw0, idw1)
        idp = (idp0, idp1)
        ttf = (ttf0, ttf1)
        tti = (tti0, tti1)
        rw = (rw0, rw1)
        rp = (rp0, rp1)
        sw = (sw0, sw1)
        sp = (sp0, sp1)

        pltpu.sync_copy(type_hbm, ty_v)
        pltpu.sync_copy(lnw_hbm, lnw_v)
        pltpu.sync_copy(lnb_hbm, lnb_v)
        for j in range(NJ):
            sl = pl.ds(j * LANES, LANES)
            td_v[sl] = ty_v[1, sl] - ty_v[0, sl]

        def issue(g, b):
            base = wbase + g * CHUNK
            pltpu.sync_copy(ids_hbm.at[pl.ds(base, CHUNK)], idw[b])
            pltpu.sync_copy(pos_hbm.at[pl.ds(base, CHUNK)], idp[b])
            pltpu.sync_copy(tt_hbm.at[pl.ds(base, CHUNK)], tti[b])
            pltpu.async_copy(word_hbm.at[idw[b]], rw[b], sw[b])
            pltpu.async_copy(post_hbm.at[idp[b]], rp[b], sp[b])

        def wait(b):
            pltpu.make_async_copy(word_hbm.at[idw[b]], rw[b], sw[b]).wait()
            pltpu.make_async_copy(post_hbm.at[idp[b]], rp[b], sp[b]).wait()

        def compute(g, b):
            rw_v, rp_v, ttf_v, tti_v = rw[b], rp[b], ttf[b], tti[b]
            for q in range(CHUNK // LANES):
                qsl = pl.ds(q * LANES, LANES)
                ttf_v[qsl] = tti_v[qsl].astype(jnp.float32)

            def tok_body(t, tc):
                g16 = t // LANES
                lane = t - g16 * LANES
                lane_v = lax.broadcast_in_dim(lane, (LANES,), ())
                ttb = ttf_v[pl.ds(g16 * LANES, LANES)].at[lane_v].get(
                    mode="promise_in_bounds")
                acc = [jnp.zeros((LANES,), jnp.float32)
                       for _ in range(NACC)]
                acc2 = [jnp.zeros((LANES,), jnp.float32)
                        for _ in range(NACC)]
                for j in range(NJ):
                    sl = pl.ds(j * LANES, LANES)
                    x = rw_v[t, sl] + rp_v[t, sl] + (ty_v[0, sl]
                                                     + ttb * td_v[sl])
                    xv_v[t, sl] = x
                    acc[j % NACC] = acc[j % NACC] + x
                    acc2[j % NACC] = acc2[j % NACC] + x * x
                a = (acc[0] + acc[1]) + (acc[2] + acc[3])
                a2 = (acc2[0] + acc2[1]) + (acc2[2] + acc2[3])
                mv = _allsum16(a) * (1.0 / D)
                ex2 = _allsum16(a2) * (1.0 / D)
                var = ex2 - mv * mv
                inv = _rsqrt16(var + LN_EPS)
                for j in range(NJ):
                    sl = pl.ds(j * LANES, LANES)
                    xn = (xv_v[t, sl] - mv) * inv
                    rw_v[t, sl] = xn * lnw_v[sl] + lnb_v[sl]
                return tc

            lax.fori_loop(0, CHUNK, tok_body, 0)
            pltpu.sync_copy(rw_v, out_hbm.at[pl.ds(wbase + g * CHUNK, CHUNK)])

        issue(0, 0)

        def pair_body(i, carry):
            g0 = 2 * i
            wait(0)
            issue(g0 + 1, 1)
            compute(g0, 0)
            wait(1)

            @pl.when(i < NCHUNK // 2 - 1)
            def _():
                issue(g0 + 2, 0)

            compute(g0 + 1, 1)
            return carry

        lax.fori_loop(0, NCHUNK // 2, pair_body, 0)

    return k(ids, pos, tt, word_table, pos_table, type_table, ln_w, ln_b)


def kernel(input_ids, token_type_ids, position_ids, word_table, pos_table,
           type_table, ln_w, ln_b):
    ids = jnp.asarray(input_ids, jnp.int32).reshape(N)
    pos = jnp.asarray(position_ids, jnp.int32).reshape(N)
    tt = jnp.asarray(token_type_ids, jnp.int32).reshape(N)
    out = _sc_embed(ids, pos, tt,
                    word_table.astype(jnp.float32),
                    pos_table.astype(jnp.float32),
                    type_table.astype(jnp.float32),
                    ln_w.astype(jnp.float32), ln_b.astype(jnp.float32))
    return out.reshape(B, L, D)
